# Initial kernel scaffold; baseline (speedup 1.0000x reference)
import jax
import jax.numpy as jnp
from jax.experimental import pallas as pl


def kernel(user, positive, negative, ui_edges, uu_edges_0, uu_edges_1, ii_edges_0, ii_edges_1, user_table, item_table, u_gcn_w0, u_gcn_b0, u_res_w0, u_res_b0, u_gcn_w1, u_gcn_b1, u_res_w1, u_res_b1, i_gcn_w0, i_gcn_b0, i_res_w0, i_res_b0, i_gcn_w1, i_gcn_b1, i_res_w1, i_res_b1):
    raise NotImplementedError("write your pallas kernel here")


# speedup vs baseline: 9.9720x; 9.9720x over previous
import functools
import jax
import jax.numpy as jnp
from jax import lax
from jax.experimental import pallas as pl
from jax.experimental.pallas import tpu as pltpu
from jax.experimental.pallas import tpu_sc as plsc

NU = 25000
NI = 25000
D = 64
T = 0.2
REG_L = 0.0001
SSL_L = 0.1
INTRA_L = 0.1
B = 4096
E_UI = 400000
E_G = 200000

NP = 25088
PT = NP // 16
CH = 128
EA = 401408
EG = 200704
CA = EA // (16 * CH)
CG = EG // (16 * CH)
RB = 512
NB = NP // RB
SZ = 112

_mesh = functools.partial(plsc.VectorSubcoreMesh,
                          core_axis_name="c", subcore_axis_name="s")
_SC_PARAMS = pltpu.CompilerParams(use_tc_tiling_on_sc=False)


def _make_hist(nch_by_job, n_idx):
    njobs = len(nch_by_job)

    @functools.partial(
        pl.kernel, mesh=_mesh(), compiler_params=_SC_PARAMS,
        out_type=[jax.ShapeDtypeStruct((NP,), jnp.float32)] * (2 * njobs),
        scratch_types=[
            pltpu.VMEM_SHARED((NP,), jnp.float32),
            pltpu.VMEM((CH,), jnp.int32),
            pltpu.VMEM((CH,), jnp.float32),
            pltpu.VMEM((PT,), jnp.float32),
            pltpu.VMEM((PT,), jnp.float32),
        ],
    )
    def k(*refs):
        z1 = refs[0]
        idxs = refs[1:1 + n_idx]
        outs = refs[1 + n_idx:1 + n_idx + 2 * njobs]
        acc, bidx, ones, zbuf, stage = refs[-5:]
        c = lax.axis_index("c")
        t = lax.axis_index("s")
        row0 = t * PT
        for i in range(CH // 16):
            ones[pl.ds(i * 16, 16)] = jnp.ones((16,), jnp.float32)
        pltpu.sync_copy(z1.at[pl.ds(0, PT)], zbuf)
        for j in range(njobs):
            pltpu.sync_copy(zbuf, acc.at[pl.ds(row0, PT)])
            plsc.subcore_barrier()
            nch = nch_by_job[j]
            for ci in range(2):
                e_i = ci * njobs + j

                @pl.when(c == ci)
                def _():
                    def body(ch, carry):
                        base = (t * nch + ch) * CH
                        pltpu.sync_copy(idxs[e_i].at[pl.ds(base, CH)], bidx)
                        pltpu.sync_copy(ones, acc.at[bidx], add=True)
                        return carry
                    lax.fori_loop(0, nch, body, 0)
            plsc.subcore_barrier()
            for ci in range(2):
                o_i = ci * njobs + j

                @pl.when(c == ci)
                def _():
                    pltpu.sync_copy(acc.at[pl.ds(row0, PT)], stage)
                    pltpu.sync_copy(stage, outs[o_i].at[pl.ds(row0, PT)])
            plsc.subcore_barrier()
    return k


def _make_segsum(jobspec, n_tables, n_edges):
    njobs = len(jobspec[0])
    n_out = sum(len(js) for js in jobspec)

    @functools.partial(
        pl.kernel, mesh=_mesh(), compiler_params=_SC_PARAMS,
        out_type=[jax.ShapeDtypeStruct((NP, D), jnp.float32)] * n_out,
        scratch_types=[
            pltpu.VMEM_SHARED((NP, D), jnp.float32),
            pltpu.VMEM((CH,), jnp.int32),
            pltpu.VMEM((CH,), jnp.int32),
            pltpu.VMEM((CH, D), jnp.float32),
            pltpu.VMEM((SZ, D), jnp.float32),
            pltpu.VMEM((SZ, D), jnp.float32),
            pltpu.SemaphoreType.DMA,
        ],
    )
    def k(*refs):
        tables = refs[:n_tables]
        z2 = refs[n_tables]
        srcs = refs[n_tables + 1:n_tables + 1 + n_edges]
        dsts = refs[n_tables + 1 + n_edges:n_tables + 1 + 2 * n_edges]
        outs = refs[n_tables + 1 + 2 * n_edges:n_tables + 1 + 2 * n_edges + n_out]
        acc, sidx, didx, rows, zbuf, stage, sem = refs[-7:]
        c = lax.axis_index("c")
        t = lax.axis_index("s")
        row0 = t * PT
        pltpu.sync_copy(z2.at[pl.ds(0, SZ)], zbuf)
        for j in range(njobs):
            for kk in range(PT // SZ):
                pltpu.sync_copy(zbuf, acc.at[pl.ds(row0 + kk * SZ, SZ)])
            plsc.subcore_barrier()
            for ci in range(2):
                tbl_i, e_i, nch, _ = jobspec[ci][j]

                @pl.when(c == ci)
                def _():
                    def body(ch, carry):
                        base = (t * nch + ch) * CH
                        pltpu.sync_copy(srcs[e_i].at[pl.ds(base, CH)], sidx)
                        pltpu.sync_copy(dsts[e_i].at[pl.ds(base, CH)], didx)
                        pltpu.async_copy(tables[tbl_i].at[sidx], rows, sem).wait()
                        pltpu.sync_copy(rows, acc.at[didx], add=True)
                        return carry
                    lax.fori_loop(0, nch, body, 0)
            plsc.subcore_barrier()
            for ci in range(2):
                o_i = jobspec[ci][j][3]

                @pl.when(c == ci)
                def _():
                    for kk in range(PT // SZ):
                        pltpu.sync_copy(acc.at[pl.ds(row0 + kk * SZ, SZ)],
                                        stage)
                        pltpu.sync_copy(stage,
                                        outs[o_i].at[pl.ds(row0 + kk * SZ, SZ)])
            plsc.subcore_barrier()
    return k


def _make_gather(jobspec, table_shapes, n_idx):
    n_out = sum(len(js) for js in jobspec)
    out_types = [None] * n_out
    for js in jobspec:
        for tbl_i, _, o_i, is_scalar in js:
            shp = (B,) if is_scalar else (B, D)
            out_types[o_i] = jax.ShapeDtypeStruct(shp, jnp.float32)
    n_tab = len(table_shapes)

    @functools.partial(
        pl.kernel, mesh=_mesh(), compiler_params=_SC_PARAMS,
        out_type=out_types,
        scratch_types=[
            pltpu.VMEM((CH,), jnp.int32),
            pltpu.VMEM((CH, D), jnp.float32),
            pltpu.VMEM((CH,), jnp.float32),
            pltpu.SemaphoreType.DMA,
        ],
    )
    def k(*refs):
        tables = refs[:n_tab]
        idxs = refs[n_tab:n_tab + n_idx]
        outs = refs[n_tab + n_idx:n_tab + n_idx + n_out]
        iidx, rows, svals, sem = refs[-4:]
        c = lax.axis_index("c")
        t = lax.axis_index("s")
        nch = B // (16 * CH)
        for ci in range(2):
            @pl.when(c == ci)
            def _():
                for tbl_i, idx_i, o_i, is_scalar in jobspec[ci]:
                    for ch in range(nch):
                        base = (t * nch + ch) * CH
                        pltpu.sync_copy(idxs[idx_i].at[pl.ds(base, CH)], iidx)
                        if is_scalar:
                            pltpu.async_copy(tables[tbl_i].at[iidx], svals,
                                             sem).wait()
                            pltpu.sync_copy(svals, outs[o_i].at[pl.ds(base, CH)])
                        else:
                            pltpu.async_copy(tables[tbl_i].at[iidx], rows,
                                             sem).wait()
                            pltpu.sync_copy(rows, outs[o_i].at[pl.ds(base, CH)])
    return k


def _prep_tc(degu, degi, dout0, dout1, dout2, dout3, ut_p, it_p,
             wg, bg, wr, br):
    def body(degu_r, degi_r, d0_r, d1_r, d2_r, d3_r, ut_r, it_r,
             wg_r, bg_r, wr_r, br_r,
             x0s_r, h0_r, h1_r, h2_r, h3_r, r0_r, r1_r, r2_r, r3_r,
             dvu_r, dvi_r):
        du = lax.rsqrt(jnp.maximum(degu_r[...], 1.0))
        di = lax.rsqrt(jnp.maximum(degi_r[...], 1.0))
        dvu_r[...] = du
        dvi_r[...] = di
        ut = ut_r[...]
        it = it_r[...]
        x0s_r[0] = ut * du
        x0s_r[1] = it * di
        douts = [d0_r, d1_r, d2_r, d3_r]
        h_outs = [h0_r, h1_r, h2_r, h3_r]
        r_outs = [r0_r, r1_r, r2_r, r3_r]
        for g in range(4):
            x = ut if g < 2 else it
            dsc = lax.rsqrt(jnp.maximum(douts[g][...], 1.0))
            h_outs[g][...] = jnp.dot(x, wg_r[g],
                                     preferred_element_type=jnp.float32) * dsc
            r_outs[g][...] = (jnp.dot(x, wr_r[g],
                                      preferred_element_type=jnp.float32)
                              + (bg_r[g] + br_r[g]))

    vb = pl.BlockSpec((RB, 1), lambda i: (i, 0))
    mb = pl.BlockSpec((RB, D), lambda i: (i, 0))
    wb = pl.BlockSpec((4, D, D), lambda i: (0, 0, 0))
    bb = pl.BlockSpec((4, D), lambda i: (0, 0))
    return pl.pallas_call(
        body,
        grid=(NB,),
        in_specs=[vb, vb, vb, vb, vb, vb, mb, mb, wb, bb, wb, bb],
        out_specs=[pl.BlockSpec((2, RB, D), lambda i: (0, i, 0)),
                   mb, mb, mb, mb, mb, mb, mb, mb, vb, vb],
        out_shape=[jax.ShapeDtypeStruct((2, NP, D), jnp.float32)]
        + [jax.ShapeDtypeStruct((NP, D), jnp.float32)] * 8
        + [jax.ShapeDtypeStruct((NP, 1), jnp.float32)] * 2,
    )(degu, degi, dout0, dout1, dout2, dout3, ut_p, it_p, wg, bg, wr, br)


def _mid_tc(aggu, aggi, dvu, dvi, g0, g1, g2, g3, r0, r1, r2, r3,
            din0, din1, din2, din3):
    def body(au_r, ai_r, dvu_r, dvi_r, g0_r, g1_r, g2_r, g3_r,
             r0_r, r1_r, r2_r, r3_r, d0_r, d1_r, d2_r, d3_r,
             x1s_r, v0_r, v1_r, v2_r, v3_r):
        du2 = dvu_r[...] ** 2
        di2 = dvi_r[...] ** 2
        x1s_r[0] = au_r[...] * du2
        x1s_r[1] = ai_r[...] * di2
        gs = [g0_r, g1_r, g2_r, g3_r]
        rs = [r0_r, r1_r, r2_r, r3_r]
        dins = [d0_r, d1_r, d2_r, d3_r]
        vouts = [v0_r, v1_r, v2_r, v3_r]
        for g in range(4):
            dsc = lax.rsqrt(jnp.maximum(dins[g][...], 1.0))
            vouts[g][...] = gs[g][...] * dsc + rs[g][...]

    vb = pl.BlockSpec((RB, 1), lambda i: (i, 0))
    mb = pl.BlockSpec((RB, D), lambda i: (i, 0))
    return pl.pallas_call(
        body,
        grid=(NB,),
        in_specs=[mb, mb, vb, vb, mb, mb, mb, mb, mb, mb, mb, mb,
                  vb, vb, vb, vb],
        out_specs=[pl.BlockSpec((2, RB, D), lambda i: (0, i, 0)),
                   mb, mb, mb, mb],
        out_shape=[jax.ShapeDtypeStruct((2, NP, D), jnp.float32)]
        + [jax.ShapeDtypeStruct((NP, D), jnp.float32)] * 4,
    )(aggu, aggi, dvu, dvi, g0, g1, g2, g3, r0, r1, r2, r3,
      din0, din1, din2, din3)


def _loss_pre_tc(r1u, r2u, hu0u, hu1u, utu, r1p, r2p, hi0p, hi1p, itp,
                 r1n, r2n, itn, dvu, dvp, dvn):
    def body(r1u_r, r2u_r, hu0_r, hu1_r, utu_r, r1p_r, r2p_r, hi0_r,
             hi1_r, itp_r, r1n_r, r2n_r, itn_r, dvu_r, dvp_r, dvn_r,
             a_r, b_r, pos_r, bsp_r, reg_r):
        i = pl.program_id(0)
        u_e = 0.5 * dvu_r[...] * (r1u_r[...] + r2u_r[...])
        p_e = 0.5 * dvp_r[...] * (r1p_r[...] + r2p_r[...])
        n_e = 0.5 * dvn_r[...] * (r1n_r[...] + r2n_r[...])
        hu0 = hu0_r[...]
        hu1 = hu1_r[...]
        hi0 = hi0_r[...]
        hi1 = hi1_r[...]
        uview = 0.5 * (hu0 + hu1)
        iview = 0.5 * (hi0 + hi1)

        def nrm(v):
            return v / (jnp.sqrt(jnp.sum(v * v, -1, keepdims=True)) + 1e-8)

        pairs = [(hu0, hu1), (hi0, hi1), (u_e, uview), (p_e, iview)]
        for g, (v1, v2) in enumerate(pairs):
            a = nrm(v1)
            b = nrm(v2)
            a_r[g] = a
            b_r[g] = b
            pos_r[g] = jnp.sum(a * b, -1, keepdims=True) / T
        bprd = (jnp.sum(u_e * n_e, -1, keepdims=True)
                - jnp.sum(u_e * p_e, -1, keepdims=True))
        bsp_r[...] = jnp.maximum(bprd, 0.0) + jnp.log(1.0
                                                      + jnp.exp(-jnp.abs(bprd)))
        utu = utu_r[...]
        itp = itp_r[...]
        itn = itn_r[...]
        regv = (REG_L * 0.5 / B) * (jnp.sum(utu * utu)
                                    + jnp.sum(itp * itp)
                                    + jnp.sum(itn * itn))

        @pl.when(i == 0)
        def _():
            reg_r[...] = jnp.zeros((1, 1), jnp.float32)
        reg_r[...] += regv.reshape(1, 1)

    nblk = B // RB
    mb = pl.BlockSpec((RB, D), lambda i: (i, 0))
    cb = pl.BlockSpec((RB, 1), lambda i: (i, 0))
    return pl.pallas_call(
        body,
        grid=(nblk,),
        in_specs=[mb] * 13 + [cb] * 3,
        out_specs=[pl.BlockSpec((4, RB, D), lambda i: (0, i, 0)),
                   pl.BlockSpec((4, RB, D), lambda i: (0, i, 0)),
                   pl.BlockSpec((4, RB, 1), lambda i: (0, i, 0)),
                   cb,
                   pl.BlockSpec((1, 1), lambda i: (0, 0))],
        out_shape=[jax.ShapeDtypeStruct((4, B, D), jnp.float32),
                   jax.ShapeDtypeStruct((4, B, D), jnp.float32),
                   jax.ShapeDtypeStruct((4, B, 1), jnp.float32),
                   jax.ShapeDtypeStruct((B, 1), jnp.float32),
                   jax.ShapeDtypeStruct((1, 1), jnp.float32)],
    )(r1u, r2u, hu0u, hu1u, utu, r1p, r2p, hi0p, hi1p, itp,
      r1n, r2n, itn, dvu, dvp, dvn)


def _nce_tc(av, bv):
    def body(a_r, b_r, ttl_r):
        i = pl.program_id(1)
        a = a_r[0]
        bm = b_r[0]
        S = lax.dot_general(a, bm, (((1,), (1,)), ((), ())),
                            preferred_element_type=jnp.float32) / T
        m = jnp.max(S, 1, keepdims=True)
        lse = m + jnp.log(jnp.sum(jnp.exp(S - m), 1, keepdims=True))
        ttl_r[0, pl.ds(i * RB, RB), :] = lse

    return pl.pallas_call(
        body,
        grid=(4, B // RB),
        in_specs=[pl.BlockSpec((1, RB, D), lambda g, i: (g, i, 0)),
                  pl.BlockSpec((1, B, D), lambda g, i: (g, 0, 0))],
        out_specs=pl.BlockSpec((1, B, 1), lambda g, i: (g, 0, 0)),
        out_shape=jax.ShapeDtypeStruct((4, B, 1), jnp.float32),
    )(av, bv)


def _loss_final_tc(ttl, pos, bsp, reg):
    def body(ttl_r, pos_r, bsp_r, reg_r, o_r):
        d = ttl_r[...] - pos_r[...]
        m0 = jnp.mean(d[0])
        m1 = jnp.mean(d[1])
        m2 = jnp.mean(d[2])
        m3 = jnp.mean(d[3])
        bpr = jnp.mean(bsp_r[...])
        reg = reg_r[0, 0]
        ssl = SSL_L * (m2 + m3)
        intra = INTRA_L * (m0 + m1)
        o_r[...] = jnp.concatenate(
            [v.reshape(1, 1) for v in (bpr, reg, ssl, intra)], axis=1)

    return pl.pallas_call(
        body,
        out_shape=jax.ShapeDtypeStruct((1, 4), jnp.float32),
    )(ttl, pos, bsp, reg)


def kernel(user, positive, negative, ui_edges, uu_edges_0, uu_edges_1,
           ii_edges_0, ii_edges_1, user_table, item_table,
           u_gcn_w0, u_gcn_b0, u_res_w0, u_res_b0,
           u_gcn_w1, u_gcn_b1, u_res_w1, u_res_b1,
           i_gcn_w0, i_gcn_b0, i_res_w0, i_res_b0,
           i_gcn_w1, i_gcn_b1, i_res_w1, i_res_b1):
    f32 = jnp.float32
    ut_p = jnp.pad(user_table, ((0, NP - NU), (0, 0)))
    it_p = jnp.pad(item_table, ((0, NP - NI), (0, 0)))
    z1 = jnp.zeros((NP,), f32)
    z2 = jnp.zeros((NP, D), f32)
    gedges = [uu_edges_0, uu_edges_1, ii_edges_0, ii_edges_1]

    pad_src = lambda n: (jnp.arange(n, dtype=jnp.int32) % 64)
    pad_dst = lambda n: NU + (jnp.arange(n, dtype=jnp.int32) % 64)
    u_idx, i_idx = ui_edges[0], ui_edges[1]
    src0 = jnp.concatenate([i_idx + NP, pad_src(EA - E_UI)])
    dst0 = jnp.concatenate([u_idx, pad_dst(EA - E_UI)])
    src1 = jnp.concatenate([u_idx, pad_src(EA - E_UI)])
    dst1 = jnp.concatenate([i_idx, pad_dst(EA - E_UI)])
    sg = [jnp.concatenate([e[0], pad_src(EG - E_G)]) for e in gedges]
    dg = [jnp.concatenate([e[1], pad_dst(EG - E_G)]) for e in gedges]

    hidx = [jnp.concatenate([u_idx, pad_dst(EA - E_UI)]),
            jnp.concatenate([gedges[0][0], pad_dst(EG - E_G)]),
            jnp.concatenate([gedges[0][1], pad_dst(EG - E_G)]),
            jnp.concatenate([gedges[2][0], pad_dst(EG - E_G)]),
            jnp.concatenate([gedges[2][1], pad_dst(EG - E_G)]),
            jnp.concatenate([i_idx, pad_dst(EA - E_UI)]),
            jnp.concatenate([gedges[1][0], pad_dst(EG - E_G)]),
            jnp.concatenate([gedges[1][1], pad_dst(EG - E_G)]),
            jnp.concatenate([gedges[3][0], pad_dst(EG - E_G)]),
            jnp.concatenate([gedges[3][1], pad_dst(EG - E_G)])]
    k1 = _make_hist((CA, CG, CG, CG, CG), 10)
    (deg_u, g0_out, g0_in, g2_out, g2_in,
     deg_i, g1_out, g1_in, g3_out, g3_in) = k1(z1, *hidx)
    douts = [g0_out, g1_out, g2_out, g3_out]
    dins = [g0_in, g1_in, g2_in, g3_in]

    v128 = lambda a: a.reshape(NP, 1)
    wg = jnp.stack([u_gcn_w0, u_gcn_w1, i_gcn_w0, i_gcn_w1])
    bg = jnp.stack([u_gcn_b0, u_gcn_b1, i_gcn_b0, i_gcn_b1])
    wr = jnp.stack([u_res_w0, u_res_w1, i_res_w0, i_res_w1])
    br = jnp.stack([u_res_b0, u_res_b1, i_res_b0, i_res_b1])
    (x0s, h0, h1, h2, h3, r0, r1, r2, r3, dvu, dvi) = _prep_tc(
        v128(deg_u), v128(deg_i), v128(douts[0]), v128(douts[1]),
        v128(douts[2]), v128(douts[3]), ut_p, it_p, wg, bg, wr, br)
    x0s_flat = x0s.reshape(2 * NP, D)

    segspec3 = [[(0, 0, CA, 0), (1, 2, CG, 2), (2, 3, CG, 3)],
                [(0, 1, CA, 1), (3, 4, CG, 4), (4, 5, CG, 5)]]
    k3 = _make_segsum(segspec3, 5, 6)
    aggu1, aggi1, g0raw, g1raw, g2raw, g3raw = k3(
        x0s_flat, h0, h1, h2, h3, z2,
        src0, src1, sg[0], sg[1], sg[2], sg[3],
        dst0, dst1, dg[0], dg[1], dg[2], dg[3])

    x1s, hv0, hv1, hv2, hv3 = _mid_tc(
        aggu1, aggi1, dvu, dvi, g0raw, g1raw, g2raw, g3raw,
        r0, r1, r2, r3,
        v128(dins[0]), v128(dins[1]), v128(dins[2]), v128(dins[3]))
    x1s_flat = x1s.reshape(2 * NP, D)

    segspec5 = [[(0, 0, CA, 0)], [(0, 1, CA, 1)]]
    k5a = _make_segsum(segspec5, 1, 2)
    aggu2, aggi2 = k5a(x1s_flat, z2, src0, src1, dst0, dst1)

    gspec = [[(0, 0, 0, False), (2, 0, 1, False), (4, 0, 2, False),
              (5, 0, 3, False), (8, 0, 4, False), (10, 0, 5, True),
              (1, 2, 6, False), (3, 2, 7, False)],
             [(1, 1, 8, False), (3, 1, 9, False), (6, 1, 10, False),
              (7, 1, 11, False), (9, 1, 12, False), (11, 1, 13, True),
              (9, 2, 14, False), (11, 2, 15, True)]]
    tables = [aggu1, aggi1, aggu2, aggi2, hv0, hv1, hv2, hv3, ut_p, it_p,
              dvu.reshape(NP), dvi.reshape(NP)]
    k5b = _make_gather(gspec, [t.shape for t in tables], 3)
    (r1u, r2u, hu0u, hu1u, utu, dvu_b, r1n, r2n,
     r1p, r2p, hi0p, hi1p, itp, dvp_b, itn, dvn_b) = k5b(
        *tables, user, positive, negative)

    v128b = lambda a: a.reshape(B, 1)
    av, bv, pos, bsp, reg = _loss_pre_tc(
        r1u, r2u, hu0u, hu1u, utu, r1p, r2p, hi0p, hi1p, itp,
        r1n, r2n, itn, v128b(dvu_b), v128b(dvp_b), v128b(dvn_b))
    ttl = _nce_tc(av, bv)
    out = _loss_final_tc(ttl, pos, bsp, reg)
    return out.reshape(4)
